# baseline (speedup 1.0000x reference)
import jax
import jax.numpy as jnp
from jax import lax
from jax.experimental import pallas as pl
from jax.experimental.pallas import tpu as pltpu
from jax.experimental.pallas import tpu_sc as plsc

V = 8192
C = 4096
E = 32768
B = 128
NITER = 5
NW = 16
CPW = C // NW
VPW = V // NW
CHUNK = 32
GPV = VPW // 2
EP = E + 2 * NW * CHUNK
EPH = EP // 2
HB = B // 2
HP = V // 2 + 8
VR = VPW // 2
INF = float("inf")


def _sld(ref, i):
    return ref[pl.ds(i, 16)][0]


def _worker_ranges(seg_of_edge, nseg, segs_per_worker, nw):
    order = jnp.argsort(seg_of_edge)
    seg_s = seg_of_edge[order].astype(jnp.int32)
    deg = jnp.zeros((nseg,), jnp.int32).at[seg_of_edge].add(1)
    off = jnp.concatenate(
        [jnp.zeros((1,), jnp.int32), jnp.cumsum(deg, dtype=jnp.int32)])
    wstart = off[::segs_per_worker]
    wlen = wstart[1:] - wstart[:-1]
    wpad = ((wlen + CHUNK - 1) // CHUNK) * CHUNK
    pstart = jnp.concatenate(
        [jnp.zeros((1,), jnp.int32), jnp.cumsum(wpad, dtype=jnp.int32)])[:nw]
    ar = jnp.arange(E, dtype=jnp.int32)
    wj = jnp.searchsorted(wstart[1:], ar, side="right").astype(jnp.int32)
    ppos = pstart[wj] + (ar - wstart[wj])
    return order, seg_s, wj, ppos, pstart, wlen


def _setup_indices(edge_var, edge_chk):
    order_c, chk_s, wj_c, ppos_c, pstart_c, wlen_c = _worker_ranges(
        edge_chk, C, CPW, NW)
    var_sc = edge_var[order_c].astype(jnp.int32)
    var_p = jnp.full((EP,), V, jnp.int32).at[ppos_c].set(var_sc)
    chk_p = jnp.zeros((EP,), jnp.int32).at[ppos_c].set(chk_s - wj_c * CPW)
    order_v, var_sv, wj_v, ppos_v, pstart_v, wlen_v = _worker_ranges(
        edge_var, V, GPV, 2 * NW)
    pos_of_edge = jnp.zeros((E,), jnp.int32).at[order_c].set(ppos_c)
    cpos_p = jnp.zeros((EP,), jnp.int32).at[ppos_v].set(pos_of_edge[order_v])
    vvid_p = jnp.zeros((EP,), jnp.int32).at[ppos_v].set(var_sv - wj_v * GPV)
    meta = jnp.concatenate(
        [pstart_c, wlen_c, pstart_v, wlen_v,
         jnp.zeros((16,), jnp.int32)]).astype(jnp.int32)
    return var_p, chk_p, cpos_p, vvid_p, meta


def _sc_body(llr_p, var_p, chk_p, cpos_p, vvid_p, meta, wch, wv, bv,
             chv_hbm, tot_hbm, mvc_hbm, mcv_hbm,
             meta_v, wab_v, bv_v, rowb, outb, gsb, chtot,
             m1s, m2s, varc, chkc, gidx, sem):
    h = lax.axis_index("c")
    w = lax.axis_index("s")
    pltpu.sync_copy(meta, meta_v)
    pltpu.sync_copy(wch, wab_v)
    pltpu.sync_copy(bv, bv_v)
    estart = pl.multiple_of(_sld(meta_v, w), CHUNK)
    elen = _sld(meta_v, NW + w)
    nchunk = (elen + (CHUNK - 1)) // CHUNK
    vb2 = pl.multiple_of(w * VR, 8)
    hrow = h * HP
    inf16 = jnp.full((16,), INF, jnp.float32)
    one16 = jnp.full((16,), 1.0, jnp.float32)
    NJ = HB // 16
    HR = CHUNK // 2

    def st_row(eb):
        return pl.ds(
            pl.multiple_of(h * EPH + lax.shift_right_logical(eb, 1), HR), HR)

    def init_a(k, _):
        rb = pl.multiple_of(vb2 + k * HR, HR)
        pltpu.sync_copy(
            llr_p.at[pl.ds(pl.multiple_of(h * (V // 2) + rb, HR), HR)], rowb)
        def row(i, _):
            s0 = _sld(wab_v, 2 * (rb + i))
            s1 = _sld(wab_v, 2 * (rb + i) + 1)
            for j in range(NJ):
                sl0 = pl.ds(j * 16, 16)
                sl1 = pl.ds(HB + j * 16, 16)
                outb[i, sl0] = rowb[i, sl0] * s0
                outb[i, sl1] = rowb[i, sl1] * s1
            return 0
        lax.fori_loop(0, HR, row, 0)
        pltpu.sync_copy(
            outb, chv_hbm.at[pl.ds(pl.multiple_of(hrow + rb, 8), HR)])
        return 0
    lax.fori_loop(0, VR // HR, init_a, 0)
    pltpu.sync_copy(wv, wab_v)
    plsc.subcore_barrier()

    def init_b(ci, _):
        eb = pl.multiple_of(estart + ci * CHUNK, CHUNK)
        pltpu.sync_copy(var_p.at[pl.ds(eb, CHUNK)], varc.at[pl.ds(0, CHUNK)])
        for j in range(CHUNK // 16):
            sl = pl.ds(j * 16, 16)
            gidx[sl] = lax.shift_right_logical(varc[sl], 1) + hrow
        pltpu.async_copy(chv_hbm.at[gidx.at[pl.ds(0, CHUNK)]], gsb, sem).wait()
        nval = jnp.minimum(elen - ci * CHUNK, CHUNK)
        def rep(i, _):
            vid = _sld(varc, i)
            src = (vid & 1) * HB
            dst = (i & 1) * HB
            ro = lax.shift_right_logical(i, 1)
            for j in range(NJ):
                outb[ro, pl.ds(dst + j * 16, 16)] = \
                    gsb[i, pl.ds(src + j * 16, 16)]
            return 0
        lax.fori_loop(0, nval, rep, 0)
        pltpu.sync_copy(outb, mvc_hbm.at[st_row(eb)])
        return 0
    lax.fori_loop(0, nchunk, init_b, 0)

    for t in range(NITER):
        def st_init(c, _):
            for j in range(NJ):
                sl = pl.ds(j * 16, 16)
                m1s[c, sl] = inf16
                m2s[c, sl] = inf16
            return 0
        lax.fori_loop(0, CPW, st_init, 0)

        def p1_chunk(ci, _):
            eb = pl.multiple_of(estart + ci * CHUNK, CHUNK)
            pltpu.sync_copy(mvc_hbm.at[st_row(eb)], rowb)
            pltpu.sync_copy(chk_p.at[pl.ds(eb, CHUNK)],
                            chkc.at[pl.ds(0, CHUNK)])
            nval = jnp.minimum(elen - ci * CHUNK, CHUNK)
            def p1_edge(i, _):
                c = _sld(chkc, i)
                ro = lax.shift_right_logical(i, 1)
                co = (i & 1) * HB
                for j in range(NJ):
                    sl = pl.ds(j * 16, 16)
                    v = rowb[ro, pl.ds(co + j * 16, 16)]
                    a = jnp.abs(v)
                    s = jnp.where(v < 0.0, -1.0, 1.0).astype(jnp.float32)
                    m1 = m1s[c, sl]
                    m2e = m2s[c, sl]
                    sg = jnp.where(m2e < 0.0, -1.0, 1.0).astype(jnp.float32)
                    m2 = jnp.abs(m2e)
                    cand = jnp.where(a < m1, m1, jnp.where(a > m1, a, inf16))
                    m1s[c, sl] = jnp.minimum(m1, a)
                    m2s[c, sl] = jnp.minimum(m2, cand) * (sg * s)
                return 0
            lax.fori_loop(0, nval, p1_edge, 0)
            return 0
        lax.fori_loop(0, nchunk, p1_chunk, 0)

        def p2_chunk(ci, _):
            eb = pl.multiple_of(estart + ci * CHUNK, CHUNK)
            pltpu.sync_copy(mvc_hbm.at[st_row(eb)], rowb)
            pltpu.sync_copy(chk_p.at[pl.ds(eb, CHUNK)],
                            chkc.at[pl.ds(0, CHUNK)])
            nval = jnp.minimum(elen - ci * CHUNK, CHUNK)
            def p2_edge(i, _):
                c = _sld(chkc, i)
                ro = lax.shift_right_logical(i, 1)
                co = (i & 1) * HB
                for j in range(NJ):
                    sl = pl.ds(j * 16, 16)
                    v = rowb[ro, pl.ds(co + j * 16, 16)]
                    a = jnp.abs(v)
                    s = jnp.where(v < 0.0, -1.0, 1.0).astype(jnp.float32)
                    m2e = m2s[c, sl]
                    sg = jnp.where(m2e < 0.0, -1.0, 1.0).astype(jnp.float32)
                    ext = jnp.where(a == m1s[c, sl], jnp.abs(m2e), m1s[c, sl])
                    ext = jnp.where(ext > 1e30, 0.0, ext)
                    outb[ro, pl.ds(co + j * 16, 16)] = sg * s * ext
                return 0
            lax.fori_loop(0, nval, p2_edge, 0)
            pltpu.sync_copy(outb, mcv_hbm.at[st_row(eb)])
            return 0
        lax.fori_loop(0, nchunk, p2_chunk, 0)
        plsc.subcore_barrier()

        for g2 in range(2):
            g = 2 * w + g2
            evstart = pl.multiple_of(_sld(meta_v, 2 * NW + g), CHUNK)
            evlen = _sld(meta_v, 4 * NW + g)
            nvchunk = (evlen + (CHUNK - 1)) // CHUNK
            sb2 = pl.multiple_of(g * (GPV // 2), 8)
            pltpu.sync_copy(
                chv_hbm.at[pl.ds(pl.multiple_of(hrow + sb2, 8), GPV // 2)],
                chtot)
            def pv_chunk(ci, _):
                eb = pl.multiple_of(evstart + ci * CHUNK, CHUNK)
                pltpu.sync_copy(cpos_p.at[pl.ds(eb, CHUNK)],
                                varc.at[pl.ds(0, CHUNK)])
                pltpu.sync_copy(vvid_p.at[pl.ds(eb, CHUNK)],
                                chkc.at[pl.ds(0, CHUNK)])
                for j in range(CHUNK // 16):
                    sl = pl.ds(j * 16, 16)
                    gidx[sl] = lax.shift_right_logical(varc[sl], 1) + h * EPH
                pltpu.async_copy(mcv_hbm.at[gidx.at[pl.ds(0, CHUNK)]], gsb,
                                 sem).wait()
                nval = jnp.minimum(evlen - ci * CHUNK, CHUNK)
                def pv_edge(i, _):
                    pos = _sld(varc, i)
                    vloc = _sld(chkc, i)
                    so = (pos & 1) * HB
                    ro = lax.shift_right_logical(vloc, 1)
                    dst = (vloc & 1) * HB
                    for j in range(NJ):
                        dsl = pl.ds(dst + j * 16, 16)
                        chtot[ro, dsl] = chtot[ro, dsl] + \
                            gsb[i, pl.ds(so + j * 16, 16)]
                    return 0
                lax.fori_loop(0, nval, pv_edge, 0)
                return 0
            lax.fori_loop(0, nvchunk, pv_chunk, 0)
            pltpu.sync_copy(
                chtot,
                tot_hbm.at[pl.ds(pl.multiple_of(hrow + sb2, 8), GPV // 2)])
        plsc.subcore_barrier()

        if t < NITER - 1:
            def p3_chunk(ci, _):
                eb = pl.multiple_of(estart + ci * CHUNK, CHUNK)
                pltpu.sync_copy(mcv_hbm.at[st_row(eb)], rowb)
                pltpu.sync_copy(var_p.at[pl.ds(eb, CHUNK)],
                                varc.at[pl.ds(0, CHUNK)])
                for j in range(CHUNK // 16):
                    sl = pl.ds(j * 16, 16)
                    gidx[sl] = lax.shift_right_logical(varc[sl], 1) + hrow
                pltpu.async_copy(tot_hbm.at[gidx.at[pl.ds(0, CHUNK)]], gsb,
                                 sem).wait()
                nval = jnp.minimum(elen - ci * CHUNK, CHUNK)
                def p3_edge(i, _):
                    vid = _sld(varc, i)
                    wsc = _sld(wab_v, vid)
                    bsc = _sld(bv_v, vid)
                    ro = lax.shift_right_logical(i, 1)
                    co = (i & 1) * HB
                    so = (vid & 1) * HB
                    for j in range(NJ):
                        g = gsb[i, pl.ds(so + j * 16, 16)]
                        mc = rowb[ro, pl.ds(co + j * 16, 16)]
                        outb[ro, pl.ds(co + j * 16, 16)] = \
                            (g - mc) * wsc + bsc
                    return 0
                lax.fori_loop(0, nval, p3_edge, 0)
                pltpu.sync_copy(outb, mvc_hbm.at[st_row(eb)])
                return 0
            lax.fori_loop(0, nchunk, p3_chunk, 0)


def kernel(llr, edge_var, edge_chk, W_vc, B_vc, W_ch):
    var_p, chk_p, cpos_p, vvid_p, meta = _setup_indices(edge_var, edge_chk)
    llr_p = llr.reshape(2, HB, V // 2, 2).transpose(0, 2, 3, 1).reshape(V, B)

    mesh = plsc.VectorSubcoreMesh(core_axis_name="c", subcore_axis_name="s")
    f32 = jnp.float32
    sc = pl.kernel(
        _sc_body,
        out_type=[
            jax.ShapeDtypeStruct((2 * HP, B), f32),
            jax.ShapeDtypeStruct((2 * HP, B), f32),
            jax.ShapeDtypeStruct((2 * EPH, B), f32),
            jax.ShapeDtypeStruct((2 * EPH, B), f32),
        ],
        mesh=mesh,
        scratch_types=[
            pltpu.VMEM((6 * NW + 16,), jnp.int32),
            pltpu.VMEM((V + 16,), f32),
            pltpu.VMEM((V + 16,), f32),
            pltpu.VMEM((CHUNK // 2, B), f32),
            pltpu.VMEM((CHUNK // 2, B), f32),
            pltpu.VMEM((CHUNK, B), f32),
            pltpu.VMEM((GPV // 2, B), f32),
            pltpu.VMEM((CPW, HB), f32),
            pltpu.VMEM((CPW, HB), f32),
            pltpu.VMEM((CHUNK + 16,), jnp.int32),
            pltpu.VMEM((CHUNK + 16,), jnp.int32),
            pltpu.VMEM((CHUNK + 16,), jnp.int32),
            pltpu.SemaphoreType.DMA,
        ],
    )
    pad16 = jnp.zeros((16,), jnp.float32)
    wch = jnp.concatenate([W_ch[0], pad16])
    wv = jnp.concatenate([W_vc[0], pad16])
    bv = jnp.concatenate([B_vc[0], pad16])
    _, tot, _, _ = sc(llr_p, var_p, chk_p, cpos_p, vvid_p, meta, wch, wv, bv)
    out_p = jnp.concatenate([tot[:V // 2], tot[HP:HP + V // 2]], axis=0)
    return out_p.reshape(2, V // 2, 2, HB).transpose(0, 3, 1, 2).reshape(B, V)


# speedup vs baseline: 1.3658x; 1.3658x over previous
import jax
import jax.numpy as jnp
from jax import lax
from jax.experimental import pallas as pl
from jax.experimental.pallas import tpu as pltpu
from jax.experimental.pallas import tpu_sc as plsc

V = 8192
C = 4096
E = 32768
B = 128
NITER = 5
NW = 16
CPW = C // NW
VPW = V // NW
CHUNK = 64
GPV = VPW // 2
EP = E + 2 * NW * CHUNK
EPH = EP // 2
HB = B // 2
HP = V // 2 + 8
VR = VPW // 2
INF = float("inf")


def _sld(ref, i):
    return ref[pl.ds(i, 16)][0]


def _worker_ranges(seg_of_edge, nseg, segs_per_worker, nw):
    order = jnp.argsort(seg_of_edge)
    seg_s = seg_of_edge[order].astype(jnp.int32)
    deg = jnp.zeros((nseg,), jnp.int32).at[seg_of_edge].add(1)
    off = jnp.concatenate(
        [jnp.zeros((1,), jnp.int32), jnp.cumsum(deg, dtype=jnp.int32)])
    wstart = off[::segs_per_worker]
    wlen = wstart[1:] - wstart[:-1]
    wpad = ((wlen + CHUNK - 1) // CHUNK) * CHUNK
    pstart = jnp.concatenate(
        [jnp.zeros((1,), jnp.int32), jnp.cumsum(wpad, dtype=jnp.int32)])[:nw]
    ar = jnp.arange(E, dtype=jnp.int32)
    wj = jnp.searchsorted(wstart[1:], ar, side="right").astype(jnp.int32)
    ppos = pstart[wj] + (ar - wstart[wj])
    return order, seg_s, wj, ppos, pstart, wlen


def _setup_indices(edge_var, edge_chk):
    order_c, chk_s, wj_c, ppos_c, pstart_c, wlen_c = _worker_ranges(
        edge_chk, C, CPW, NW)
    var_sc = edge_var[order_c].astype(jnp.int32)
    var_p = jnp.full((EP,), V, jnp.int32).at[ppos_c].set(var_sc)
    chk_p = jnp.zeros((EP,), jnp.int32).at[ppos_c].set(chk_s - wj_c * CPW)
    order_v, var_sv, wj_v, ppos_v, pstart_v, wlen_v = _worker_ranges(
        edge_var, V, GPV, 2 * NW)
    pos_of_edge = jnp.zeros((E,), jnp.int32).at[order_c].set(ppos_c)
    cpos_p = jnp.zeros((EP,), jnp.int32).at[ppos_v].set(pos_of_edge[order_v])
    vvid_p = jnp.zeros((EP,), jnp.int32).at[ppos_v].set(var_sv - wj_v * GPV)
    meta = jnp.concatenate(
        [pstart_c, wlen_c, pstart_v, wlen_v,
         jnp.zeros((16,), jnp.int32)]).astype(jnp.int32)
    return var_p, chk_p, cpos_p, vvid_p, meta


def _sc_body(llr_p, var_p, chk_p, cpos_p, vvid_p, meta, wch, wv, bv,
             chv_hbm, tot_hbm, mvc_hbm, mcv_hbm,
             meta_v, wab_v, bv_v, rowb, outb, gsb, chtot,
             m1s, m2s, varc, chkc, gidx, sem):
    h = lax.axis_index("c")
    w = lax.axis_index("s")
    pltpu.sync_copy(meta, meta_v)
    pltpu.sync_copy(wch, wab_v)
    pltpu.sync_copy(bv, bv_v)
    estart = pl.multiple_of(_sld(meta_v, w), CHUNK)
    elen = _sld(meta_v, NW + w)
    nchunk = (elen + (CHUNK - 1)) // CHUNK
    vb2 = pl.multiple_of(w * VR, 8)
    hrow = h * HP
    inf16 = jnp.full((16,), INF, jnp.float32)
    one16 = jnp.full((16,), 1.0, jnp.float32)
    NJ = HB // 16
    HR = CHUNK // 2

    def st_row(eb):
        return pl.ds(
            pl.multiple_of(h * EPH + lax.shift_right_logical(eb, 1), HR), HR)

    def init_a(k, _):
        rb = pl.multiple_of(vb2 + k * HR, HR)
        pltpu.sync_copy(
            llr_p.at[pl.ds(pl.multiple_of(h * (V // 2) + rb, HR), HR)], rowb)
        @plsc.parallel_loop(0, HR, unroll=4)
        def row(i):
            s0 = _sld(wab_v, 2 * (rb + i))
            s1 = _sld(wab_v, 2 * (rb + i) + 1)
            for j in range(NJ):
                sl0 = pl.ds(j * 16, 16)
                sl1 = pl.ds(HB + j * 16, 16)
                outb[i, sl0] = rowb[i, sl0] * s0
                outb[i, sl1] = rowb[i, sl1] * s1
        pltpu.sync_copy(
            outb, chv_hbm.at[pl.ds(pl.multiple_of(hrow + rb, 8), HR)])
        return 0
    lax.fori_loop(0, VR // HR, init_a, 0)
    pltpu.sync_copy(wv, wab_v)
    plsc.subcore_barrier()

    def init_b(ci, _):
        eb = pl.multiple_of(estart + ci * CHUNK, CHUNK)
        pltpu.sync_copy(var_p.at[pl.ds(eb, CHUNK)], varc.at[pl.ds(0, CHUNK)])
        for j in range(CHUNK // 16):
            sl = pl.ds(j * 16, 16)
            gidx[sl] = lax.shift_right_logical(varc[sl], 1) + hrow
        pltpu.async_copy(chv_hbm.at[gidx.at[pl.ds(0, CHUNK)]], gsb, sem).wait()
        nval = jnp.minimum(elen - ci * CHUNK, CHUNK)
        @plsc.parallel_loop(0, nval, unroll=4)
        def rep(i):
            vid = _sld(varc, i)
            so = (vid & 1) * HB
            dst = (i & 1) * HB
            ro = lax.shift_right_logical(i, 1)
            for j in range(NJ):
                outb[ro, pl.ds(dst + j * 16, 16)] = \
                    gsb[i, pl.ds(so + j * 16, 16)]
        pltpu.sync_copy(outb, mvc_hbm.at[st_row(eb)])
        return 0
    lax.fori_loop(0, nchunk, init_b, 0)

    for t in range(NITER):
        @plsc.parallel_loop(0, CPW, unroll=4)
        def st_init(c):
            for j in range(NJ):
                sl = pl.ds(j * 16, 16)
                m1s[c, sl] = inf16
                m2s[c, sl] = inf16

        def p1_chunk(ci, _):
            eb = pl.multiple_of(estart + ci * CHUNK, CHUNK)
            pltpu.sync_copy(mvc_hbm.at[st_row(eb)], rowb)
            pltpu.sync_copy(chk_p.at[pl.ds(eb, CHUNK)],
                            chkc.at[pl.ds(0, CHUNK)])
            nval = jnp.minimum(elen - ci * CHUNK, CHUNK)
            def p1_edge(i, _):
                c = _sld(chkc, i)
                ro = lax.shift_right_logical(i, 1)
                co = (i & 1) * HB
                for j in range(NJ):
                    sl = pl.ds(j * 16, 16)
                    v = rowb[ro, pl.ds(co + j * 16, 16)]
                    a = jnp.abs(v)
                    s = jnp.where(v < 0.0, -1.0, 1.0).astype(jnp.float32)
                    m1 = m1s[c, sl]
                    m2e = m2s[c, sl]
                    sg = jnp.where(m2e < 0.0, -1.0, 1.0).astype(jnp.float32)
                    m2 = jnp.abs(m2e)
                    cand = jnp.where(a < m1, m1, jnp.where(a > m1, a, inf16))
                    m1s[c, sl] = jnp.minimum(m1, a)
                    m2s[c, sl] = jnp.minimum(m2, cand) * (sg * s)
                return 0
            lax.fori_loop(0, nval, p1_edge, 0)
            return 0
        lax.fori_loop(0, nchunk, p1_chunk, 0)

        def p2_chunk(ci, _):
            eb = pl.multiple_of(estart + ci * CHUNK, CHUNK)
            pltpu.sync_copy(mvc_hbm.at[st_row(eb)], rowb)
            pltpu.sync_copy(chk_p.at[pl.ds(eb, CHUNK)],
                            chkc.at[pl.ds(0, CHUNK)])
            nval = jnp.minimum(elen - ci * CHUNK, CHUNK)
            @plsc.parallel_loop(0, nval, unroll=4)
            def p2_edge(i):
                c = _sld(chkc, i)
                ro = lax.shift_right_logical(i, 1)
                co = (i & 1) * HB
                for j in range(NJ):
                    sl = pl.ds(j * 16, 16)
                    v = rowb[ro, pl.ds(co + j * 16, 16)]
                    a = jnp.abs(v)
                    s = jnp.where(v < 0.0, -1.0, 1.0).astype(jnp.float32)
                    m2e = m2s[c, sl]
                    sg = jnp.where(m2e < 0.0, -1.0, 1.0).astype(jnp.float32)
                    ext = jnp.where(a == m1s[c, sl], jnp.abs(m2e), m1s[c, sl])
                    ext = jnp.where(ext > 1e30, 0.0, ext)
                    outb[ro, pl.ds(co + j * 16, 16)] = sg * s * ext
            pltpu.sync_copy(outb, mcv_hbm.at[st_row(eb)])
            return 0
        lax.fori_loop(0, nchunk, p2_chunk, 0)
        plsc.subcore_barrier()

        for g2 in range(2):
            g = 2 * w + g2
            evstart = pl.multiple_of(_sld(meta_v, 2 * NW + g), CHUNK)
            evlen = _sld(meta_v, 4 * NW + g)
            nvchunk = (evlen + (CHUNK - 1)) // CHUNK
            sb2 = pl.multiple_of(g * (GPV // 2), 8)
            pltpu.sync_copy(
                chv_hbm.at[pl.ds(pl.multiple_of(hrow + sb2, 8), GPV // 2)],
                chtot)
            def pv_chunk(ci, _):
                eb = pl.multiple_of(evstart + ci * CHUNK, CHUNK)
                pltpu.sync_copy(cpos_p.at[pl.ds(eb, CHUNK)],
                                varc.at[pl.ds(0, CHUNK)])
                pltpu.sync_copy(vvid_p.at[pl.ds(eb, CHUNK)],
                                chkc.at[pl.ds(0, CHUNK)])
                for j in range(CHUNK // 16):
                    sl = pl.ds(j * 16, 16)
                    gidx[sl] = lax.shift_right_logical(varc[sl], 1) + h * EPH
                pltpu.async_copy(mcv_hbm.at[gidx.at[pl.ds(0, CHUNK)]], gsb,
                                 sem).wait()
                nval = jnp.minimum(evlen - ci * CHUNK, CHUNK)
                def pv_edge(i, _):
                    pos = _sld(varc, i)
                    vloc = _sld(chkc, i)
                    so = (pos & 1) * HB
                    ro = lax.shift_right_logical(vloc, 1)
                    dst = (vloc & 1) * HB
                    for j in range(NJ):
                        dsl = pl.ds(dst + j * 16, 16)
                        chtot[ro, dsl] = chtot[ro, dsl] + \
                            gsb[i, pl.ds(so + j * 16, 16)]
                    return 0
                lax.fori_loop(0, nval, pv_edge, 0)
                return 0
            lax.fori_loop(0, nvchunk, pv_chunk, 0)
            pltpu.sync_copy(
                chtot,
                tot_hbm.at[pl.ds(pl.multiple_of(hrow + sb2, 8), GPV // 2)])
        plsc.subcore_barrier()

        if t < NITER - 1:
            def p3_chunk(ci, _):
                eb = pl.multiple_of(estart + ci * CHUNK, CHUNK)
                pltpu.sync_copy(mcv_hbm.at[st_row(eb)], rowb)
                pltpu.sync_copy(var_p.at[pl.ds(eb, CHUNK)],
                                varc.at[pl.ds(0, CHUNK)])
                for j in range(CHUNK // 16):
                    sl = pl.ds(j * 16, 16)
                    gidx[sl] = lax.shift_right_logical(varc[sl], 1) + hrow
                pltpu.async_copy(tot_hbm.at[gidx.at[pl.ds(0, CHUNK)]], gsb,
                                 sem).wait()
                nval = jnp.minimum(elen - ci * CHUNK, CHUNK)
                @plsc.parallel_loop(0, nval, unroll=4)
                def p3_edge(i):
                    vid = _sld(varc, i)
                    wsc = _sld(wab_v, vid)
                    bsc = _sld(bv_v, vid)
                    ro = lax.shift_right_logical(i, 1)
                    co = (i & 1) * HB
                    so = (vid & 1) * HB
                    for j in range(NJ):
                        g = gsb[i, pl.ds(so + j * 16, 16)]
                        mc = rowb[ro, pl.ds(co + j * 16, 16)]
                        outb[ro, pl.ds(co + j * 16, 16)] = \
                            (g - mc) * wsc + bsc
                pltpu.sync_copy(outb, mvc_hbm.at[st_row(eb)])
                return 0
            lax.fori_loop(0, nchunk, p3_chunk, 0)


def kernel(llr, edge_var, edge_chk, W_vc, B_vc, W_ch):
    var_p, chk_p, cpos_p, vvid_p, meta = _setup_indices(edge_var, edge_chk)
    llr_p = llr.reshape(2, HB, V // 2, 2).transpose(0, 2, 3, 1).reshape(V, B)

    mesh = plsc.VectorSubcoreMesh(core_axis_name="c", subcore_axis_name="s")
    f32 = jnp.float32
    sc = pl.kernel(
        _sc_body,
        out_type=[
            jax.ShapeDtypeStruct((2 * HP, B), f32),
            jax.ShapeDtypeStruct((2 * HP, B), f32),
            jax.ShapeDtypeStruct((2 * EPH, B), f32),
            jax.ShapeDtypeStruct((2 * EPH, B), f32),
        ],
        mesh=mesh,
        scratch_types=[
            pltpu.VMEM((6 * NW + 16,), jnp.int32),
            pltpu.VMEM((V + 16,), f32),
            pltpu.VMEM((V + 16,), f32),
            pltpu.VMEM((CHUNK // 2, B), f32),
            pltpu.VMEM((CHUNK // 2, B), f32),
            pltpu.VMEM((CHUNK, B), f32),
            pltpu.VMEM((GPV // 2, B), f32),
            pltpu.VMEM((CPW, HB), f32),
            pltpu.VMEM((CPW, HB), f32),
            pltpu.VMEM((CHUNK + 16,), jnp.int32),
            pltpu.VMEM((CHUNK + 16,), jnp.int32),
            pltpu.VMEM((CHUNK + 16,), jnp.int32),
            pltpu.SemaphoreType.DMA,
        ],
    )
    pad16 = jnp.zeros((16,), jnp.float32)
    wch = jnp.concatenate([W_ch[0], pad16])
    wv = jnp.concatenate([W_vc[0], pad16])
    bv = jnp.concatenate([B_vc[0], pad16])
    _, tot, _, _ = sc(llr_p, var_p, chk_p, cpos_p, vvid_p, meta, wch, wv, bv)
    out_p = jnp.concatenate([tot[:V // 2], tot[HP:HP + V // 2]], axis=0)
    return out_p.reshape(2, V // 2, 2, HB).transpose(0, 3, 1, 2).reshape(B, V)
